# baseline (speedup 1.0000x reference)
import functools

import jax
import jax.numpy as jnp
from jax import lax
from jax.experimental import pallas as pl
from jax.experimental.pallas import tpu as pltpu
from jax.experimental.pallas import tpu_sc as plsc

_N = 2048
_NW = 32
_RPW = _N // _NW
_L = 16
_K = _N // 2
_NV = _K // _L
_CUTOFF = 5.0

_SEC = _N + _K + _L
_OX, _OY, _OZ = 0 * _SEC, 1 * _SEC, 2 * _SEC
_ORE, _OBE, _OAL = 3 * _SEC, 4 * _SEC, 5 * _SEC
_OFE, _OA, _OB = 6 * _SEC, 7 * _SEC, 8 * _SEC
_OKA, _OLA = 9 * _SEC, 10 * _SEC
_FLAT = 11 * _SEC


def _pow20(x):
    x2 = x * x
    x4 = x2 * x2
    x8 = x4 * x4
    x16 = x8 * x8
    return x16 * x4


def _rsqrt_newton(r2):
    bits = lax.bitcast_convert_type(r2, jnp.int32)
    seed = jnp.int32(0x5F3759DF) - lax.shift_right_logical(bits, 1)
    y = lax.bitcast_convert_type(seed, jnp.float32)
    half = -0.5 * r2
    for _ in range(3):
        y = y * (1.5 + half * y * y)
    return y


def _sc_body(flat_hbm, rhor_hbm, rhoc_hbm, pairs_hbm, data, rho_v, rhoc, pair_v):
    wid = lax.axis_index("s") * 2 + lax.axis_index("c")
    base = wid * _RPW

    pltpu.sync_copy(flat_hbm, data)

    def zero_body(k, _):
        rhoc[pl.ds(k * _L, _L)] = jnp.zeros((_L,), jnp.float32)
        return 0
    lax.fori_loop(0, 2 * _N // _L, zero_body, 0)

    def _sload(off):
        return data[pl.ds(off, _L)][0]

    def row_body(il, pair_carry):
        i = base + il
        xi = _sload(_OX + i)
        yi = _sload(_OY + i)
        zi = _sload(_OZ + i)
        ire_i = _sload(_ORE + i)
        be_i = _sload(_OBE + i)
        al_i = _sload(_OAL + i)
        fe_i = _sload(_OFE + i)
        a_i = _sload(_OA + i)
        bofe_i = _sload(_OB + i)
        ka_i = _sload(_OKA + i)
        la_i = _sload(_OLA + i)

        def load_cols(j0):
            return (
                data[pl.ds(_OX + j0, _L)],
                data[pl.ds(_OY + j0, _L)],
                data[pl.ds(_OZ + j0, _L)],
                data[pl.ds(_ORE + j0, _L)],
                data[pl.ds(_OBE + j0, _L)],
                data[pl.ds(_OAL + j0, _L)],
                data[pl.ds(_OFE + j0, _L)],
                data[pl.ds(_OA + j0, _L)],
                data[pl.ds(_OB + j0, _L)],
                data[pl.ds(_OKA + j0, _L)],
                data[pl.ds(_OLA + j0, _L)],
            )

        def endpoint(r, ire, be, al, fe, a, bofe, ka, la):
            u = r * ire
            om = 1.0 - u
            eb = jnp.exp(om * be)
            ea = jnp.exp(om * al)
            dlam_s = (1.0 + _pow20(u - la)) * (2.0 ** -30)
            dkap = 1.0 + _pow20(u - ka)
            iprod = 1.0 / (dlam_s * dkap)
            fs = (fe * eb) * (iprod * dkap)
            phir = (a * ea) * (iprod * dlam_s) - bofe * fs
            return fs, phir

        def compute(cols):
            xj, yj, zj, ire_j, be_j, al_j, fe_j, a_j, bofe_j, ka_j, la_j = cols
            dx = xj - xi
            dy = yj - yi
            dz = zj - zi
            r2 = dx * dx + dy * dy + dz * dz
            r = r2 * _rsqrt_newton(r2)

            fs_i, phir_i = endpoint(r, ire_i, be_i, al_i, fe_i, a_i,
                                    bofe_i, ka_i, la_i)
            fs_j, phir_j = endpoint(r, ire_j, be_j, al_j, fe_j, a_j,
                                    bofe_j, ka_j, la_j)

            phi = ((fs_j * fs_j) * phir_i + (fs_i * fs_i) * phir_j) \
                / (fs_i * fs_j)
            return jnp.where(r <= _CUTOFF, phi, 0.0), fs_j, fs_i

        def col_block(m, carry):
            pair_acc, rho_acc = carry
            j0 = i + 1 + m * (3 * _L)
            cols = [load_cols(j0 + u * _L) for u in range(3)]
            res = [compute(c) for c in cols]
            for phi_m, fs_j, _ in res:
                pair_acc = pair_acc + phi_m
                rho_acc = rho_acc + fs_j
            for u, (_, _, fs_i_u) in enumerate(res):
                plsc.addupdate(rhoc.at[pl.ds(j0 + u * _L, _L)], fs_i_u)
            return pair_acc, rho_acc

        zero = jnp.zeros((_L,), jnp.float32)
        pair_acc, rho_acc = lax.fori_loop(0, 21, col_block, (zero, zero))

        klane = 1009 + lax.iota(jnp.int32, _L)
        wt = jnp.where(klane == _K, 0.5, 1.0)
        phi_m, fs_j, fs_i_t = compute(load_cols(i + 1009))
        pair_acc = pair_acc + phi_m * wt
        rho_acc = rho_acc + fs_j * wt
        plsc.addupdate(rhoc.at[pl.ds(i + 1009, _L)], fs_i_t * wt)

        rho_v[pl.ds(il * _L, _L)] = rho_acc * (2.0 ** -40)
        return pair_carry + pair_acc

    pair_tot = lax.fori_loop(0, _RPW, row_body, jnp.zeros((_L,), jnp.float32))
    pair_v[...] = pair_tot

    pltpu.sync_copy(rho_v, rhor_hbm.at[pl.ds(base * _L, _RPW * _L)])
    pltpu.sync_copy(rhoc, rhoc_hbm.at[wid])
    pltpu.sync_copy(pair_v, pairs_hbm.at[wid])


_sc_pairs = functools.partial(
    pl.kernel,
    out_type=(
        jax.ShapeDtypeStruct((_N * _L,), jnp.float32),
        jax.ShapeDtypeStruct((_NW, 2 * _N), jnp.float32),
        jax.ShapeDtypeStruct((_NW, _L), jnp.float32),
    ),
    mesh=plsc.VectorSubcoreMesh(core_axis_name="c", subcore_axis_name="s"),
    scratch_types=[
        pltpu.VMEM((_FLAT,), jnp.float32),
        pltpu.VMEM((_RPW * _L,), jnp.float32),
        pltpu.VMEM((2 * _N,), jnp.float32),
        pltpu.VMEM((_L,), jnp.float32),
    ],
)(_sc_body)


def _tc_tail_kernel(rhor_ref, rhoc_ref, pt_ref, pairs_ref, out_ref):
    rhoc = jnp.sum(rhoc_ref[...], axis=0, keepdims=True)
    rho = (jnp.sum(rhor_ref[...], axis=0, keepdims=True)
           + (rhoc[:, :_N] + rhoc[:, _N:]) * (2.0 ** -40))
    f_n0 = pt_ref[10:11, :]
    f_n1 = pt_ref[11:12, :]
    f_n2 = pt_ref[12:13, :]
    f_n3 = pt_ref[13:14, :]
    f_0 = pt_ref[14:15, :]
    f_1 = pt_ref[15:16, :]
    f_2 = pt_ref[16:17, :]
    f_3 = pt_ref[17:18, :]
    f_e = pt_ref[19:20, :]
    rho_n = pt_ref[20:21, :]
    rho_e = pt_ref[2:3, :]
    rho_0 = pt_ref[21:22, :]
    rho_s = pt_ref[3:4, :]
    eta = pt_ref[18:19, :]

    t_n = rho / rho_n - 1.0
    b1 = f_n0 + (f_n1 + (f_n2 + f_n3 * t_n) * t_n) * t_n
    t_e = rho / rho_e - 1.0
    b2 = f_0 + (f_1 + (f_2 + f_3 * t_e) * t_e) * t_e
    ratio = rho / rho_s
    lpw = eta * jnp.log(ratio)
    pw = jnp.exp(lpw)
    b3 = f_e * (1.0 - lpw) * pw
    f_val = jnp.where(rho < rho_n, b1, jnp.where(rho < rho_0, b2, b3))

    total = (jnp.sum(f_val, axis=(0, 1), keepdims=True)
             + 0.5 * jnp.sum(pairs_ref[...], axis=(0, 1), keepdims=True))
    out_ref[...] = total


def _wrap(col):
    return jnp.concatenate([col, col[: _K + _L]])


def kernel(weights, params):
    flat = jnp.concatenate([
        _wrap(weights[:, 0]), _wrap(weights[:, 1]), _wrap(weights[:, 2]),
        _wrap(1.0 / params[:, 0]), _wrap(params[:, 5]), _wrap(params[:, 4]),
        _wrap(params[:, 1] * 2.0 ** 10), _wrap(params[:, 6]),
        _wrap(params[:, 7] / params[:, 1] * 2.0 ** -40),
        _wrap(params[:, 8]), _wrap(params[:, 9]),
    ])
    rho_r, rho_c, pairs = _sc_pairs(flat)
    rho_rt = rho_r.reshape(_N, _L).T

    pt = params.T
    out = pl.pallas_call(
        _tc_tail_kernel,
        in_specs=[
            pl.BlockSpec((_L, _N), lambda: (0, 0)),
            pl.BlockSpec((_NW, 2 * _N), lambda: (0, 0)),
            pl.BlockSpec((22, _N), lambda: (0, 0)),
            pl.BlockSpec((_NW, _L), lambda: (0, 0)),
        ],
        out_specs=pl.BlockSpec((1, 1), lambda: (0, 0)),
        out_shape=jax.ShapeDtypeStruct((1, 1), jnp.float32),
    )(rho_rt, rho_c, pt, pairs)
    return out.reshape(())


# speedup vs baseline: 1.1152x; 1.1152x over previous
import functools

import jax
import jax.numpy as jnp
from jax import lax
from jax.experimental import pallas as pl
from jax.experimental.pallas import tpu as pltpu
from jax.experimental.pallas import tpu_sc as plsc

_N = 2048
_NW = 32
_RPW = _N // _NW
_L = 16
_K = _N // 2
_NV = _K // _L
_CUTOFF = 5.0

_SEC = _N + _K + _L
_OX, _OY, _OZ = 0 * _SEC, 1 * _SEC, 2 * _SEC
_ORE, _OBE, _OAL = 3 * _SEC, 4 * _SEC, 5 * _SEC
_OFE, _OA, _OB = 6 * _SEC, 7 * _SEC, 8 * _SEC
_OKA, _OLA = 9 * _SEC, 10 * _SEC
_FLAT = 11 * _SEC


def _pow20(x):
    x2 = x * x
    x4 = x2 * x2
    x8 = x4 * x4
    x16 = x8 * x8
    return x16 * x4


def _rsqrt_newton(r2):
    bits = lax.bitcast_convert_type(r2, jnp.int32)
    seed = jnp.int32(0x5F3759DF) - lax.shift_right_logical(bits, 1)
    y = lax.bitcast_convert_type(seed, jnp.float32)
    half = -0.5 * r2
    for _ in range(3):
        y = y * (1.5 + half * y * y)
    return y


def _sc_body(flat_hbm, rhor_hbm, rhoc_hbm, pairs_hbm, data, rho_v, rhoc, pair_v):
    wid = lax.axis_index("s") * 2 + lax.axis_index("c")
    base = wid * _RPW

    pltpu.sync_copy(flat_hbm, data)

    def zero_body(k, _):
        rhoc[pl.ds(k * _L, _L)] = jnp.zeros((_L,), jnp.float32)
        return 0
    lax.fori_loop(0, 2 * _N // _L, zero_body, 0)

    def _sload(off):
        return data[pl.ds(off, _L)][0]

    def row_body(il, pair_carry):
        i = base + il
        xi = _sload(_OX + i)
        yi = _sload(_OY + i)
        zi = _sload(_OZ + i)
        ire_i = _sload(_ORE + i)
        be_i = _sload(_OBE + i)
        al_i = _sload(_OAL + i)
        fe_i = _sload(_OFE + i)
        a_i = _sload(_OA + i)
        bofe_i = _sload(_OB + i)
        ka_i = _sload(_OKA + i)
        la_i = _sload(_OLA + i)

        def load_cols(j0):
            return (
                data[pl.ds(_OX + j0, _L)],
                data[pl.ds(_OY + j0, _L)],
                data[pl.ds(_OZ + j0, _L)],
                data[pl.ds(_ORE + j0, _L)],
                data[pl.ds(_OBE + j0, _L)],
                data[pl.ds(_OAL + j0, _L)],
                data[pl.ds(_OFE + j0, _L)],
                data[pl.ds(_OA + j0, _L)],
                data[pl.ds(_OB + j0, _L)],
                data[pl.ds(_OKA + j0, _L)],
                data[pl.ds(_OLA + j0, _L)],
            )

        def endpoint(r, ire, be, al, fe, a, bofe, ka, la):
            u = r * ire
            om = 1.0 - u
            eb = jnp.exp(om * be)
            ea = jnp.exp(om * al)
            dlam = 1.0 + _pow20(u - la)
            idkap = 1.0 / (1.0 + _pow20(u - ka))
            fr = fe * eb / dlam
            phir = a * ea * idkap - bofe * fr
            return fr, phir

        def compute(cols):
            xj, yj, zj, ire_j, be_j, al_j, fe_j, a_j, bofe_j, ka_j, la_j = cols
            dx = xj - xi
            dy = yj - yi
            dz = zj - zi
            r2 = dx * dx + dy * dy + dz * dz
            r = r2 * _rsqrt_newton(r2)

            fr_i, phir_i = endpoint(r, ire_i, be_i, al_i, fe_i, a_i,
                                    bofe_i, ka_i, la_i)
            fr_j, phir_j = endpoint(r, ire_j, be_j, al_j, fe_j, a_j,
                                    bofe_j, ka_j, la_j)

            phi = (fr_j / fr_i) * phir_i + (fr_i / fr_j) * phir_j
            return jnp.where(r <= _CUTOFF, phi, 0.0), fr_j, fr_i

        def col_block4(m, carry):
            pair_acc, rho_acc = carry
            j0 = i + 1 + m * (4 * _L)
            cols = [load_cols(j0 + u * _L) for u in range(4)]
            res = [compute(c) for c in cols]
            for phi_m, fr_j, _ in res:
                pair_acc = pair_acc + phi_m
                rho_acc = rho_acc + fr_j
            for u, (_, _, fr_i_u) in enumerate(res):
                plsc.addupdate(rhoc.at[pl.ds(j0 + u * _L, _L)], fr_i_u)
            return pair_acc, rho_acc

        zero = jnp.zeros((_L,), jnp.float32)
        pair_acc, rho_acc = lax.fori_loop(0, 15, col_block4, (zero, zero))

        jt = i + 961
        tcols = [load_cols(jt + u * _L) for u in range(4)]
        tres = [compute(c) for c in tcols]
        klane = 1009 + lax.iota(jnp.int32, _L)
        wt = jnp.where(klane == _K, 0.5, 1.0)
        for u, (phi_m, fr_j, _) in enumerate(tres):
            w = wt if u == 3 else 1.0
            pair_acc = pair_acc + phi_m * w
            rho_acc = rho_acc + fr_j * w
        for u, (_, _, fr_i_u) in enumerate(tres):
            w = wt if u == 3 else 1.0
            plsc.addupdate(rhoc.at[pl.ds(jt + u * _L, _L)], fr_i_u * w)

        rho_v[pl.ds(il * _L, _L)] = rho_acc
        return pair_carry + pair_acc

    pair_tot = lax.fori_loop(0, _RPW, row_body, jnp.zeros((_L,), jnp.float32))
    pair_v[...] = pair_tot

    pltpu.sync_copy(rho_v, rhor_hbm.at[pl.ds(base * _L, _RPW * _L)])
    pltpu.sync_copy(rhoc, rhoc_hbm.at[wid])
    pltpu.sync_copy(pair_v, pairs_hbm.at[wid])


_sc_pairs = functools.partial(
    pl.kernel,
    out_type=(
        jax.ShapeDtypeStruct((_N * _L,), jnp.float32),
        jax.ShapeDtypeStruct((_NW, 2 * _N), jnp.float32),
        jax.ShapeDtypeStruct((_NW, _L), jnp.float32),
    ),
    mesh=plsc.VectorSubcoreMesh(core_axis_name="c", subcore_axis_name="s"),
    scratch_types=[
        pltpu.VMEM((_FLAT,), jnp.float32),
        pltpu.VMEM((_RPW * _L,), jnp.float32),
        pltpu.VMEM((2 * _N,), jnp.float32),
        pltpu.VMEM((_L,), jnp.float32),
    ],
)(_sc_body)


def _tc_tail_kernel(rhor_ref, rhoc_ref, pt_ref, pairs_ref, out_ref):
    rhoc = jnp.sum(rhoc_ref[...], axis=0, keepdims=True)
    rho = (jnp.sum(rhor_ref[...], axis=0, keepdims=True)
           + rhoc[:, :_N] + rhoc[:, _N:])
    f_n0 = pt_ref[10:11, :]
    f_n1 = pt_ref[11:12, :]
    f_n2 = pt_ref[12:13, :]
    f_n3 = pt_ref[13:14, :]
    f_0 = pt_ref[14:15, :]
    f_1 = pt_ref[15:16, :]
    f_2 = pt_ref[16:17, :]
    f_3 = pt_ref[17:18, :]
    f_e = pt_ref[19:20, :]
    rho_n = pt_ref[20:21, :]
    rho_e = pt_ref[2:3, :]
    rho_0 = pt_ref[21:22, :]
    rho_s = pt_ref[3:4, :]
    eta = pt_ref[18:19, :]

    t_n = rho / rho_n - 1.0
    b1 = f_n0 + (f_n1 + (f_n2 + f_n3 * t_n) * t_n) * t_n
    t_e = rho / rho_e - 1.0
    b2 = f_0 + (f_1 + (f_2 + f_3 * t_e) * t_e) * t_e
    ratio = rho / rho_s
    lpw = eta * jnp.log(ratio)
    pw = jnp.exp(lpw)
    b3 = f_e * (1.0 - lpw) * pw
    f_val = jnp.where(rho < rho_n, b1, jnp.where(rho < rho_0, b2, b3))

    total = (jnp.sum(f_val, axis=(0, 1), keepdims=True)
             + 0.5 * jnp.sum(pairs_ref[...], axis=(0, 1), keepdims=True))
    out_ref[...] = total


def _wrap(col):
    return jnp.concatenate([col, col[: _K + _L]])


def kernel(weights, params):
    flat = jnp.concatenate([
        _wrap(weights[:, 0]), _wrap(weights[:, 1]), _wrap(weights[:, 2]),
        _wrap(1.0 / params[:, 0]), _wrap(params[:, 5]), _wrap(params[:, 4]),
        _wrap(params[:, 1]), _wrap(params[:, 6]),
        _wrap(params[:, 7] / params[:, 1]),
        _wrap(params[:, 8]), _wrap(params[:, 9]),
    ])
    rho_r, rho_c, pairs = _sc_pairs(flat)
    rho_rt = rho_r.reshape(_N, _L).T

    pt = params.T
    out = pl.pallas_call(
        _tc_tail_kernel,
        in_specs=[
            pl.BlockSpec((_L, _N), lambda: (0, 0)),
            pl.BlockSpec((_NW, 2 * _N), lambda: (0, 0)),
            pl.BlockSpec((22, _N), lambda: (0, 0)),
            pl.BlockSpec((_NW, _L), lambda: (0, 0)),
        ],
        out_specs=pl.BlockSpec((1, 1), lambda: (0, 0)),
        out_shape=jax.ShapeDtypeStruct((1, 1), jnp.float32),
    )(rho_rt, rho_c, pt, pairs)
    return out.reshape(())


# speedup vs baseline: 1.1750x; 1.0537x over previous
import functools

import jax
import jax.numpy as jnp
from jax import lax
from jax.experimental import pallas as pl
from jax.experimental.pallas import tpu as pltpu
from jax.experimental.pallas import tpu_sc as plsc

_N = 2048
_NW = 32
_RPW = _N // _NW
_L = 16
_K = _N // 2
_NV = _K // _L
_CUTOFF = 5.0

_SEC = _N + _K + _L
_OX, _OY, _OZ = 0 * _SEC, 1 * _SEC, 2 * _SEC
_ORE, _OBE, _OAL = 3 * _SEC, 4 * _SEC, 5 * _SEC
_OFE, _OA, _OB = 6 * _SEC, 7 * _SEC, 8 * _SEC
_OKA, _OLA = 9 * _SEC, 10 * _SEC
_FLAT = 11 * _SEC


def _pow20(x):
    x2 = x * x
    x4 = x2 * x2
    x8 = x4 * x4
    x16 = x8 * x8
    return x16 * x4


def _rsqrt_newton(r2):
    bits = lax.bitcast_convert_type(r2, jnp.int32)
    seed = jnp.int32(0x5F3759DF) - lax.shift_right_logical(bits, 1)
    y = lax.bitcast_convert_type(seed, jnp.float32)
    half = -0.5 * r2
    for _ in range(3):
        y = y * (1.5 + half * y * y)
    return y


def _sc_body(flat_hbm, rhor_hbm, rhoc_hbm, pairs_hbm, data, rho_v, rhoc, pair_v):
    wid = lax.axis_index("s") * 2 + lax.axis_index("c")
    base = wid * _RPW

    pltpu.sync_copy(flat_hbm, data)

    def zero_body(k, _):
        rhoc[pl.ds(k * _L, _L)] = jnp.zeros((_L,), jnp.float32)
        return 0
    lax.fori_loop(0, 2 * _N // _L, zero_body, 0)

    def _sload(off):
        return data[pl.ds(off, _L)][0]

    def row_body(il, pair_carry):
        i = base + il
        xi = _sload(_OX + i)
        yi = _sload(_OY + i)
        zi = _sload(_OZ + i)
        ire_i = _sload(_ORE + i)
        be_i = _sload(_OBE + i)
        al_i = _sload(_OAL + i)
        fe_i = _sload(_OFE + i)
        a_i = _sload(_OA + i)
        bofe_i = _sload(_OB + i)
        ka_i = _sload(_OKA + i)
        la_i = _sload(_OLA + i)

        def load_cols(j0):
            return (
                data[pl.ds(_OX + j0, _L)],
                data[pl.ds(_OY + j0, _L)],
                data[pl.ds(_OZ + j0, _L)],
                data[pl.ds(_ORE + j0, _L)],
                data[pl.ds(_OBE + j0, _L)],
                data[pl.ds(_OAL + j0, _L)],
                data[pl.ds(_OFE + j0, _L)],
                data[pl.ds(_OA + j0, _L)],
                data[pl.ds(_OB + j0, _L)],
                data[pl.ds(_OKA + j0, _L)],
                data[pl.ds(_OLA + j0, _L)],
            )

        def endpoint(r, ire, be, al, fe, a, bofe, ka, la):
            u = r * ire
            om = 1.0 - u
            eb = jnp.exp(om * be)
            ea = jnp.exp(om * al)
            dlam = 1.0 + _pow20(u - la)
            idkap = 1.0 / (1.0 + _pow20(u - ka))
            fr = fe * eb / dlam
            phir = a * ea * idkap - bofe * fr
            return fr, phir

        def compute(cols):
            xj, yj, zj, ire_j, be_j, al_j, fe_j, a_j, bofe_j, ka_j, la_j = cols
            dx = xj - xi
            dy = yj - yi
            dz = zj - zi
            r2 = dx * dx + dy * dy + dz * dz
            r = r2 * _rsqrt_newton(r2)

            fr_i, phir_i = endpoint(r, ire_i, be_i, al_i, fe_i, a_i,
                                    bofe_i, ka_i, la_i)
            fr_j, phir_j = endpoint(r, ire_j, be_j, al_j, fe_j, a_j,
                                    bofe_j, ka_j, la_j)

            phi = (fr_j / fr_i) * phir_i + (fr_i / fr_j) * phir_j
            return jnp.where(r <= _CUTOFF, phi, 0.0), fr_j, fr_i

        def col_block4(m, carry):
            pair_acc, rho_acc = carry
            j0 = i + 1 + m * (5 * _L)
            cols = [load_cols(j0 + u * _L) for u in range(5)]
            res = [compute(c) for c in cols]
            for phi_m, fr_j, _ in res:
                pair_acc = pair_acc + phi_m
                rho_acc = rho_acc + fr_j
            for u, (_, _, fr_i_u) in enumerate(res):
                plsc.addupdate(rhoc.at[pl.ds(j0 + u * _L, _L)], fr_i_u)
            return pair_acc, rho_acc

        zero = jnp.zeros((_L,), jnp.float32)
        pair_acc, rho_acc = lax.fori_loop(0, 12, col_block4, (zero, zero))

        jt = i + 961
        tcols = [load_cols(jt + u * _L) for u in range(4)]
        tres = [compute(c) for c in tcols]
        klane = 1009 + lax.iota(jnp.int32, _L)
        wt = jnp.where(klane == _K, 0.5, 1.0)
        for u, (phi_m, fr_j, _) in enumerate(tres):
            w = wt if u == 3 else 1.0
            pair_acc = pair_acc + phi_m * w
            rho_acc = rho_acc + fr_j * w
        for u, (_, _, fr_i_u) in enumerate(tres):
            w = wt if u == 3 else 1.0
            plsc.addupdate(rhoc.at[pl.ds(jt + u * _L, _L)], fr_i_u * w)

        rho_v[pl.ds(il * _L, _L)] = rho_acc
        return pair_carry + pair_acc

    pair_tot = lax.fori_loop(0, _RPW, row_body, jnp.zeros((_L,), jnp.float32))
    pair_v[...] = pair_tot

    pltpu.sync_copy(rho_v, rhor_hbm.at[pl.ds(base * _L, _RPW * _L)])
    pltpu.sync_copy(rhoc, rhoc_hbm.at[wid])
    pltpu.sync_copy(pair_v, pairs_hbm.at[wid])


_sc_pairs = functools.partial(
    pl.kernel,
    out_type=(
        jax.ShapeDtypeStruct((_N * _L,), jnp.float32),
        jax.ShapeDtypeStruct((_NW, 2 * _N), jnp.float32),
        jax.ShapeDtypeStruct((_NW, _L), jnp.float32),
    ),
    mesh=plsc.VectorSubcoreMesh(core_axis_name="c", subcore_axis_name="s"),
    scratch_types=[
        pltpu.VMEM((_FLAT,), jnp.float32),
        pltpu.VMEM((_RPW * _L,), jnp.float32),
        pltpu.VMEM((2 * _N,), jnp.float32),
        pltpu.VMEM((_L,), jnp.float32),
    ],
)(_sc_body)


def _tc_tail_kernel(rhor_ref, rhoc_ref, pt_ref, pairs_ref, out_ref):
    rhoc = jnp.sum(rhoc_ref[...], axis=0, keepdims=True)
    rho = (jnp.sum(rhor_ref[...], axis=0, keepdims=True)
           + rhoc[:, :_N] + rhoc[:, _N:])
    f_n0 = pt_ref[10:11, :]
    f_n1 = pt_ref[11:12, :]
    f_n2 = pt_ref[12:13, :]
    f_n3 = pt_ref[13:14, :]
    f_0 = pt_ref[14:15, :]
    f_1 = pt_ref[15:16, :]
    f_2 = pt_ref[16:17, :]
    f_3 = pt_ref[17:18, :]
    f_e = pt_ref[19:20, :]
    rho_n = pt_ref[20:21, :]
    rho_e = pt_ref[2:3, :]
    rho_0 = pt_ref[21:22, :]
    rho_s = pt_ref[3:4, :]
    eta = pt_ref[18:19, :]

    t_n = rho / rho_n - 1.0
    b1 = f_n0 + (f_n1 + (f_n2 + f_n3 * t_n) * t_n) * t_n
    t_e = rho / rho_e - 1.0
    b2 = f_0 + (f_1 + (f_2 + f_3 * t_e) * t_e) * t_e
    ratio = rho / rho_s
    lpw = eta * jnp.log(ratio)
    pw = jnp.exp(lpw)
    b3 = f_e * (1.0 - lpw) * pw
    f_val = jnp.where(rho < rho_n, b1, jnp.where(rho < rho_0, b2, b3))

    total = (jnp.sum(f_val, axis=(0, 1), keepdims=True)
             + 0.5 * jnp.sum(pairs_ref[...], axis=(0, 1), keepdims=True))
    out_ref[...] = total


def _wrap(col):
    return jnp.concatenate([col, col[: _K + _L]])


def kernel(weights, params):
    flat = jnp.concatenate([
        _wrap(weights[:, 0]), _wrap(weights[:, 1]), _wrap(weights[:, 2]),
        _wrap(1.0 / params[:, 0]), _wrap(params[:, 5]), _wrap(params[:, 4]),
        _wrap(params[:, 1]), _wrap(params[:, 6]),
        _wrap(params[:, 7] / params[:, 1]),
        _wrap(params[:, 8]), _wrap(params[:, 9]),
    ])
    rho_r, rho_c, pairs = _sc_pairs(flat)
    rho_rt = rho_r.reshape(_N, _L).T

    pt = params.T
    out = pl.pallas_call(
        _tc_tail_kernel,
        in_specs=[
            pl.BlockSpec((_L, _N), lambda: (0, 0)),
            pl.BlockSpec((_NW, 2 * _N), lambda: (0, 0)),
            pl.BlockSpec((22, _N), lambda: (0, 0)),
            pl.BlockSpec((_NW, _L), lambda: (0, 0)),
        ],
        out_specs=pl.BlockSpec((1, 1), lambda: (0, 0)),
        out_shape=jax.ShapeDtypeStruct((1, 1), jnp.float32),
    )(rho_rt, rho_c, pt, pairs)
    return out.reshape(())


# speedup vs baseline: 1.2161x; 1.0350x over previous
import functools

import jax
import jax.numpy as jnp
from jax import lax
from jax.experimental import pallas as pl
from jax.experimental.pallas import tpu as pltpu
from jax.experimental.pallas import tpu_sc as plsc

_N = 2048
_NW = 32
_RPW = _N // _NW
_L = 16
_K = _N // 2
_NV = _K // _L
_CUTOFF = 5.0

_SEC = _N + _K + _L
_OX, _OY, _OZ = 0 * _SEC, 1 * _SEC, 2 * _SEC
_ORE, _OBE, _OAL = 3 * _SEC, 4 * _SEC, 5 * _SEC
_OFE, _OA, _OB = 6 * _SEC, 7 * _SEC, 8 * _SEC
_OKA, _OLA = 9 * _SEC, 10 * _SEC
_FLAT = 11 * _SEC


def _pow20(x):
    x2 = x * x
    x4 = x2 * x2
    x8 = x4 * x4
    x16 = x8 * x8
    return x16 * x4


def _rsqrt_newton(r2):
    bits = lax.bitcast_convert_type(r2, jnp.int32)
    seed = jnp.int32(0x5F3759DF) - lax.shift_right_logical(bits, 1)
    y = lax.bitcast_convert_type(seed, jnp.float32)
    half = -0.5 * r2
    for _ in range(3):
        y = y * (1.5 + half * y * y)
    return y


def _sc_body(flat_hbm, rhor_hbm, rhoc_hbm, pairs_hbm, data, rho_v, rhoc, pair_v):
    wid = lax.axis_index("s") * 2 + lax.axis_index("c")
    base = wid * _RPW

    pltpu.sync_copy(flat_hbm, data)

    def zero_body(k, _):
        rhoc[pl.ds(k * _L, _L)] = jnp.zeros((_L,), jnp.float32)
        return 0
    lax.fori_loop(0, 2 * _N // _L, zero_body, 0)

    def _sload(off):
        return data[pl.ds(off, _L)][0]

    def row_body(il, pair_carry):
        i = base + il
        xi = _sload(_OX + i)
        yi = _sload(_OY + i)
        zi = _sload(_OZ + i)
        ire_i = _sload(_ORE + i)
        be_i = _sload(_OBE + i)
        al_i = _sload(_OAL + i)
        fe_i = _sload(_OFE + i)
        a_i = _sload(_OA + i)
        bofe_i = _sload(_OB + i)
        ka_i = _sload(_OKA + i)
        la_i = _sload(_OLA + i)

        def load_cols(j0):
            return (
                data[pl.ds(_OX + j0, _L)],
                data[pl.ds(_OY + j0, _L)],
                data[pl.ds(_OZ + j0, _L)],
                data[pl.ds(_ORE + j0, _L)],
                data[pl.ds(_OBE + j0, _L)],
                data[pl.ds(_OAL + j0, _L)],
                data[pl.ds(_OFE + j0, _L)],
                data[pl.ds(_OA + j0, _L)],
                data[pl.ds(_OB + j0, _L)],
                data[pl.ds(_OKA + j0, _L)],
                data[pl.ds(_OLA + j0, _L)],
            )

        def endpoint(r, ire, be, al, fe, a, bofe, ka, la):
            u = r * ire
            om = 1.0 - u
            eb = jnp.exp(om * be)
            ea = jnp.exp(om * al)
            dlam = 1.0 + _pow20(u - la)
            idkap = 1.0 / (1.0 + _pow20(u - ka))
            fr = fe * eb / dlam
            phir = a * ea * idkap - bofe * fr
            return fr, phir

        def compute(cols):
            xj, yj, zj, ire_j, be_j, al_j, fe_j, a_j, bofe_j, ka_j, la_j = cols
            dx = xj - xi
            dy = yj - yi
            dz = zj - zi
            r2 = dx * dx + dy * dy + dz * dz
            r = r2 * _rsqrt_newton(r2)

            fr_i, phir_i = endpoint(r, ire_i, be_i, al_i, fe_i, a_i,
                                    bofe_i, ka_i, la_i)
            fr_j, phir_j = endpoint(r, ire_j, be_j, al_j, fe_j, a_j,
                                    bofe_j, ka_j, la_j)

            phi = (fr_j / fr_i) * phir_i + (fr_i / fr_j) * phir_j
            return jnp.where(r <= _CUTOFF, phi, 0.0), fr_j, fr_i

        def col_block4(m, carry):
            pair_acc, rho_acc = carry
            j0 = i + 1 + m * (6 * _L)
            cols = [load_cols(j0 + u * _L) for u in range(6)]
            res = [compute(c) for c in cols]
            for phi_m, fr_j, _ in res:
                pair_acc = pair_acc + phi_m
                rho_acc = rho_acc + fr_j
            for u, (_, _, fr_i_u) in enumerate(res):
                plsc.addupdate(rhoc.at[pl.ds(j0 + u * _L, _L)], fr_i_u)
            return pair_acc, rho_acc

        zero = jnp.zeros((_L,), jnp.float32)
        pair_acc, rho_acc = lax.fori_loop(0, 10, col_block4, (zero, zero))

        jt = i + 961
        tcols = [load_cols(jt + u * _L) for u in range(4)]
        tres = [compute(c) for c in tcols]
        klane = 1009 + lax.iota(jnp.int32, _L)
        wt = jnp.where(klane == _K, 0.5, 1.0)
        for u, (phi_m, fr_j, _) in enumerate(tres):
            w = wt if u == 3 else 1.0
            pair_acc = pair_acc + phi_m * w
            rho_acc = rho_acc + fr_j * w
        for u, (_, _, fr_i_u) in enumerate(tres):
            w = wt if u == 3 else 1.0
            plsc.addupdate(rhoc.at[pl.ds(jt + u * _L, _L)], fr_i_u * w)

        rho_v[pl.ds(il * _L, _L)] = rho_acc
        return pair_carry + pair_acc

    pair_tot = lax.fori_loop(0, _RPW, row_body, jnp.zeros((_L,), jnp.float32))
    pair_v[...] = pair_tot

    pltpu.sync_copy(rho_v, rhor_hbm.at[pl.ds(base * _L, _RPW * _L)])
    pltpu.sync_copy(rhoc, rhoc_hbm.at[wid])
    pltpu.sync_copy(pair_v, pairs_hbm.at[wid])


_sc_pairs = functools.partial(
    pl.kernel,
    out_type=(
        jax.ShapeDtypeStruct((_N * _L,), jnp.float32),
        jax.ShapeDtypeStruct((_NW, 2 * _N), jnp.float32),
        jax.ShapeDtypeStruct((_NW, _L), jnp.float32),
    ),
    mesh=plsc.VectorSubcoreMesh(core_axis_name="c", subcore_axis_name="s"),
    scratch_types=[
        pltpu.VMEM((_FLAT,), jnp.float32),
        pltpu.VMEM((_RPW * _L,), jnp.float32),
        pltpu.VMEM((2 * _N,), jnp.float32),
        pltpu.VMEM((_L,), jnp.float32),
    ],
)(_sc_body)


def _tc_tail_kernel(rhor_ref, rhoc_ref, pt_ref, pairs_ref, out_ref):
    rhoc = jnp.sum(rhoc_ref[...], axis=0, keepdims=True)
    rho = (jnp.sum(rhor_ref[...], axis=0, keepdims=True)
           + rhoc[:, :_N] + rhoc[:, _N:])
    f_n0 = pt_ref[10:11, :]
    f_n1 = pt_ref[11:12, :]
    f_n2 = pt_ref[12:13, :]
    f_n3 = pt_ref[13:14, :]
    f_0 = pt_ref[14:15, :]
    f_1 = pt_ref[15:16, :]
    f_2 = pt_ref[16:17, :]
    f_3 = pt_ref[17:18, :]
    f_e = pt_ref[19:20, :]
    rho_n = pt_ref[20:21, :]
    rho_e = pt_ref[2:3, :]
    rho_0 = pt_ref[21:22, :]
    rho_s = pt_ref[3:4, :]
    eta = pt_ref[18:19, :]

    t_n = rho / rho_n - 1.0
    b1 = f_n0 + (f_n1 + (f_n2 + f_n3 * t_n) * t_n) * t_n
    t_e = rho / rho_e - 1.0
    b2 = f_0 + (f_1 + (f_2 + f_3 * t_e) * t_e) * t_e
    ratio = rho / rho_s
    lpw = eta * jnp.log(ratio)
    pw = jnp.exp(lpw)
    b3 = f_e * (1.0 - lpw) * pw
    f_val = jnp.where(rho < rho_n, b1, jnp.where(rho < rho_0, b2, b3))

    total = (jnp.sum(f_val, axis=(0, 1), keepdims=True)
             + 0.5 * jnp.sum(pairs_ref[...], axis=(0, 1), keepdims=True))
    out_ref[...] = total


def _wrap(col):
    return jnp.concatenate([col, col[: _K + _L]])


def kernel(weights, params):
    flat = jnp.concatenate([
        _wrap(weights[:, 0]), _wrap(weights[:, 1]), _wrap(weights[:, 2]),
        _wrap(1.0 / params[:, 0]), _wrap(params[:, 5]), _wrap(params[:, 4]),
        _wrap(params[:, 1]), _wrap(params[:, 6]),
        _wrap(params[:, 7] / params[:, 1]),
        _wrap(params[:, 8]), _wrap(params[:, 9]),
    ])
    rho_r, rho_c, pairs = _sc_pairs(flat)
    rho_rt = rho_r.reshape(_N, _L).T

    pt = params.T
    out = pl.pallas_call(
        _tc_tail_kernel,
        in_specs=[
            pl.BlockSpec((_L, _N), lambda: (0, 0)),
            pl.BlockSpec((_NW, 2 * _N), lambda: (0, 0)),
            pl.BlockSpec((22, _N), lambda: (0, 0)),
            pl.BlockSpec((_NW, _L), lambda: (0, 0)),
        ],
        out_specs=pl.BlockSpec((1, 1), lambda: (0, 0)),
        out_shape=jax.ShapeDtypeStruct((1, 1), jnp.float32),
    )(rho_rt, rho_c, pt, pairs)
    return out.reshape(())
